# baseline (speedup 1.0000x reference)
import functools

import jax
import jax.numpy as jnp
from jax import lax
from jax.experimental import pallas as pl
from jax.experimental.pallas import tpu as pltpu
from jax.experimental.pallas import tpu_sc as plsc

NC, NS = 2, 16
NW = NC * NS
B = 16384
D = 64
BPW = B // NW
H1, H2 = 128, 64
CHUNK = 2048
IB = 128
NI = BPW // IB
LANES = 16


def _gather_body(uid_hbm, iid_hbm, utab_hbm, itab_hbm, ubias_hbm, ibias_hbm,
                 uemb_out, iemb_out, bsum_out,
                 uidx_v, iidx_v, urow_v, irow_v,
                 urows_v, irows_v, ubrows_v, ibrows_v, bsum_v, sem):
    wid = lax.axis_index("s") * NC + lax.axis_index("c")
    base = wid * BPW
    pltpu.sync_copy(uid_hbm.at[pl.ds(wid * NI, NI)], uidx_v)
    pltpu.sync_copy(iid_hbm.at[pl.ds(wid * NI, NI)], iidx_v)
    for j in range(NI):
        for k in range(IB // LANES):
            sl = pl.ds(k * LANES, LANES)
            urow_v[j, sl] = uidx_v[j, sl] >> 4
            irow_v[j, sl] = iidx_v[j, sl] >> 4
    cps = []
    for j in range(NI):
        sl = pl.ds(j * IB, IB)
        cps.append(pltpu.async_copy(utab_hbm.at[uidx_v.at[j]], urows_v.at[sl], sem))
        cps.append(pltpu.async_copy(itab_hbm.at[iidx_v.at[j]], irows_v.at[sl], sem))
        cps.append(pltpu.async_copy(ubias_hbm.at[urow_v.at[j]], ubrows_v.at[sl], sem))
        cps.append(pltpu.async_copy(ibias_hbm.at[irow_v.at[j]], ibrows_v.at[sl], sem))
    for cp in cps:
        cp.wait()
    for c in range(BPW // LANES):
        j, k = divmod(c, IB // LANES)
        sl = pl.ds(k * LANES, LANES)
        pos = lax.iota(jnp.int32, LANES) + c * LANES
        ubv16 = plsc.load_gather(ubrows_v, [pos, uidx_v[j, sl] & 15])
        ibv16 = plsc.load_gather(ibrows_v, [pos, iidx_v[j, sl] & 15])
        bsum_v[pl.ds(c * LANES, LANES)] = ubv16 + ibv16
    pltpu.sync_copy(urows_v, uemb_out.at[pl.ds(base, BPW)])
    pltpu.sync_copy(irows_v, iemb_out.at[pl.ds(base, BPW)])
    pltpu.sync_copy(bsum_v, bsum_out.at[pl.ds(base, BPW)])


@functools.cache
def _make_gather():
    return functools.partial(
        pl.kernel,
        out_type=[
            jax.ShapeDtypeStruct((B, D), jnp.float32),
            jax.ShapeDtypeStruct((B, D), jnp.float32),
            jax.ShapeDtypeStruct((B,), jnp.float32),
        ],
        mesh=plsc.VectorSubcoreMesh(core_axis_name="c", subcore_axis_name="s"),
        compiler_params=pltpu.CompilerParams(use_tc_tiling_on_sc=False,
                                             needs_layout_passes=False),
        scratch_types=[
            pltpu.VMEM((NI, IB), jnp.int32),
            pltpu.VMEM((NI, IB), jnp.int32),
            pltpu.VMEM((NI, IB), jnp.int32),
            pltpu.VMEM((NI, IB), jnp.int32),
            pltpu.VMEM((BPW, D), jnp.float32),
            pltpu.VMEM((BPW, D), jnp.float32),
            pltpu.VMEM((BPW, LANES), jnp.float32),
            pltpu.VMEM((BPW, LANES), jnp.float32),
            pltpu.VMEM((BPW,), jnp.float32),
            pltpu.SemaphoreType.DMA,
        ],
    )(_gather_body)


def _mlp_body(u_ref, i_ref, b_ref, w1u_ref, w1i_ref, b1_ref,
              w2_ref, b2_ref, w3_ref, b3_ref, o_ref):
    h = jnp.dot(u_ref[...], w1u_ref[...], preferred_element_type=jnp.float32)
    h = h + jnp.dot(i_ref[...], w1i_ref[...], preferred_element_type=jnp.float32)
    h = jnp.maximum(h + b1_ref[...], 0.0)
    h = jnp.maximum(
        jnp.dot(h, w2_ref[...], preferred_element_type=jnp.float32) + b2_ref[...],
        0.0)
    o = jnp.sum(h * w3_ref[...], axis=1, keepdims=True)
    o_ref[...] = o + b3_ref[...] + b_ref[...]


def _mlp(uemb, iemb, bsum, w1u, w1i, b1, w2, b2, w3, b3):
    full = lambda b: (0, 0)
    return pl.pallas_call(
        _mlp_body,
        grid=(B // CHUNK,),
        in_specs=[
            pl.BlockSpec((CHUNK, D), lambda b: (b, 0)),
            pl.BlockSpec((CHUNK, D), lambda b: (b, 0)),
            pl.BlockSpec((CHUNK, 1), lambda b: (b, 0)),
            pl.BlockSpec((D, H1), full),
            pl.BlockSpec((D, H1), full),
            pl.BlockSpec((1, H1), full),
            pl.BlockSpec((H1, H2), full),
            pl.BlockSpec((1, H2), full),
            pl.BlockSpec((1, H2), full),
            pl.BlockSpec((1, 1), full),
        ],
        out_specs=pl.BlockSpec((CHUNK, 1), lambda b: (b, 0)),
        out_shape=jax.ShapeDtypeStruct((B, 1), jnp.float32),
    )(uemb, iemb, bsum, w1u, w1i, b1, w2, b2, w3, b3)


def kernel(user_ids, item_ids, user_table, item_table, user_bias, item_bias,
           W1, b1, W2, b2, W3, b3):
    ubp = jnp.pad(user_bias.reshape(-1), (0, 15)).reshape(-1, LANES)
    ibp = jnp.pad(item_bias.reshape(-1), (0, 15)).reshape(-1, LANES)
    uemb, iemb, bsum = _make_gather()(
        user_ids.astype(jnp.int32).reshape(B // IB, IB),
        item_ids.astype(jnp.int32).reshape(B // IB, IB),
        user_table, item_table, ubp, ibp)
    w1u = W1[:, :D].T
    w1i = W1[:, D:].T
    out = _mlp(uemb, iemb, bsum.reshape(B, 1), w1u, w1i, b1.reshape(1, H1),
               W2.T, b2.reshape(1, H2), W3, b3.reshape(1, 1))
    return out[:, 0]


# speedup vs baseline: 1.0006x; 1.0006x over previous
import functools

import jax
import jax.numpy as jnp
from jax import lax
from jax.experimental import pallas as pl
from jax.experimental.pallas import tpu as pltpu
from jax.experimental.pallas import tpu_sc as plsc

NC, NS = 2, 16
NW = NC * NS
B = 16384
D = 64
BPW = B // NW
H1, H2 = 128, 64
CHUNK = 2048
IB = 128
NI = BPW // IB
LANES = 16


def _gather_body(uid_hbm, iid_hbm, utab_hbm, itab_hbm, ubias_hbm, ibias_hbm,
                 uemb_out, iemb_out, bsum_out,
                 uidx_v, iidx_v, urow_v, irow_v,
                 urows_v, irows_v, ubrows_v, ibrows_v, bsum_v, sem):
    wid = lax.axis_index("s") * NC + lax.axis_index("c")
    base = wid * BPW
    pltpu.sync_copy(uid_hbm.at[pl.ds(wid * NI, NI)], uidx_v)
    pltpu.sync_copy(iid_hbm.at[pl.ds(wid * NI, NI)], iidx_v)
    for j in range(NI):
        for k in range(IB // LANES):
            sl = pl.ds(k * LANES, LANES)
            urow_v[j, sl] = uidx_v[j, sl] >> 4
            irow_v[j, sl] = iidx_v[j, sl] >> 4
    cps = []
    for j in range(NI):
        sl = pl.ds(j * IB, IB)
        cps.append(pltpu.async_copy(utab_hbm.at[uidx_v.at[j]], urows_v.at[sl], sem))
        cps.append(pltpu.async_copy(itab_hbm.at[iidx_v.at[j]], irows_v.at[sl], sem))
        cps.append(pltpu.async_copy(ubias_hbm.at[urow_v.at[j]], ubrows_v.at[sl], sem))
        cps.append(pltpu.async_copy(ibias_hbm.at[irow_v.at[j]], ibrows_v.at[sl], sem))
    for cp in cps:
        cp.wait()
    for c in range(BPW // LANES):
        j, k = divmod(c, IB // LANES)
        sl = pl.ds(k * LANES, LANES)
        pos = lax.iota(jnp.int32, LANES) + c * LANES
        ubv16 = plsc.load_gather(ubrows_v, [pos, uidx_v[j, sl] & 15])
        ibv16 = plsc.load_gather(ibrows_v, [pos, iidx_v[j, sl] & 15])
        bsum_v[pl.ds(c * LANES, LANES)] = ubv16 + ibv16
    pltpu.sync_copy(urows_v, uemb_out.at[pl.ds(base, BPW)])
    pltpu.sync_copy(irows_v, iemb_out.at[pl.ds(base, BPW)])
    pltpu.sync_copy(bsum_v, bsum_out.at[pl.ds(base, BPW)])


@functools.cache
def _make_gather():
    return functools.partial(
        pl.kernel,
        out_type=[
            jax.ShapeDtypeStruct((B, D), jnp.float32),
            jax.ShapeDtypeStruct((B, D), jnp.float32),
            jax.ShapeDtypeStruct((B,), jnp.float32),
        ],
        mesh=plsc.VectorSubcoreMesh(core_axis_name="c", subcore_axis_name="s"),
        compiler_params=pltpu.CompilerParams(use_tc_tiling_on_sc=False,
                                             needs_layout_passes=False),
        scratch_types=[
            pltpu.VMEM((NI, IB), jnp.int32),
            pltpu.VMEM((NI, IB), jnp.int32),
            pltpu.VMEM((NI, IB), jnp.int32),
            pltpu.VMEM((NI, IB), jnp.int32),
            pltpu.VMEM((BPW, D), jnp.float32),
            pltpu.VMEM((BPW, D), jnp.float32),
            pltpu.VMEM((BPW, LANES), jnp.float32),
            pltpu.VMEM((BPW, LANES), jnp.float32),
            pltpu.VMEM((BPW,), jnp.float32),
            pltpu.SemaphoreType.DMA,
        ],
    )(_gather_body)


NB = NUM_ROWS = 1000001
PADW = 62501 * LANES
RCHUNK = 131072


def _repack_body(ub_ref, ib_ref, ubo_ref, ibo_ref):
    ubo_ref[...] = ub_ref[...]
    ibo_ref[...] = ib_ref[...]


def _repack(ubf, ibf):
    ngrid = (PADW + RCHUNK - 1) // RCHUNK
    return pl.pallas_call(
        _repack_body,
        grid=(ngrid,),
        in_specs=[pl.BlockSpec((RCHUNK,), lambda b: (b,)),
                  pl.BlockSpec((RCHUNK,), lambda b: (b,))],
        out_specs=[pl.BlockSpec((RCHUNK,), lambda b: (b,)),
                   pl.BlockSpec((RCHUNK,), lambda b: (b,))],
        out_shape=[jax.ShapeDtypeStruct((PADW,), jnp.float32),
                   jax.ShapeDtypeStruct((PADW,), jnp.float32)],
    )(ubf, ibf)


def _mlp_body(u_ref, i_ref, b_ref, w1u_ref, w1i_ref, b1_ref,
              w2_ref, b2_ref, w3_ref, b3_ref, o_ref):
    h = jnp.dot(u_ref[...], w1u_ref[...], preferred_element_type=jnp.float32)
    h = h + jnp.dot(i_ref[...], w1i_ref[...], preferred_element_type=jnp.float32)
    h = jnp.maximum(h + b1_ref[...], 0.0)
    h = jnp.maximum(
        jnp.dot(h, w2_ref[...], preferred_element_type=jnp.float32) + b2_ref[...],
        0.0)
    o = jnp.sum(h * w3_ref[...], axis=1, keepdims=True)
    o_ref[...] = o + b3_ref[...] + b_ref[...]


def _mlp(uemb, iemb, bsum, w1u, w1i, b1, w2, b2, w3, b3):
    full = lambda b: (0, 0)
    return pl.pallas_call(
        _mlp_body,
        grid=(B // CHUNK,),
        in_specs=[
            pl.BlockSpec((CHUNK, D), lambda b: (b, 0)),
            pl.BlockSpec((CHUNK, D), lambda b: (b, 0)),
            pl.BlockSpec((CHUNK, 1), lambda b: (b, 0)),
            pl.BlockSpec((D, H1), full),
            pl.BlockSpec((D, H1), full),
            pl.BlockSpec((1, H1), full),
            pl.BlockSpec((H1, H2), full),
            pl.BlockSpec((1, H2), full),
            pl.BlockSpec((1, H2), full),
            pl.BlockSpec((1, 1), full),
        ],
        out_specs=pl.BlockSpec((CHUNK, 1), lambda b: (b, 0)),
        out_shape=jax.ShapeDtypeStruct((B, 1), jnp.float32),
    )(uemb, iemb, bsum, w1u, w1i, b1, w2, b2, w3, b3)


def kernel(user_ids, item_ids, user_table, item_table, user_bias, item_bias,
           W1, b1, W2, b2, W3, b3):
    ubp, ibp = _repack(user_bias.reshape(-1), item_bias.reshape(-1))
    ubp = ubp.reshape(-1, LANES)
    ibp = ibp.reshape(-1, LANES)
    uemb, iemb, bsum = _make_gather()(
        user_ids.astype(jnp.int32).reshape(B // IB, IB),
        item_ids.astype(jnp.int32).reshape(B // IB, IB),
        user_table, item_table, ubp, ibp)
    w1u = W1[:, :D].T
    w1i = W1[:, D:].T
    out = _mlp(uemb, iemb, bsum.reshape(B, 1), w1u, w1i, b1.reshape(1, H1),
               W2.T, b2.reshape(1, H2), W3, b3.reshape(1, 1))
    return out[:, 0]


# speedup vs baseline: 1.0138x; 1.0132x over previous
import functools

import jax
import jax.numpy as jnp
from jax import lax
from jax.experimental import pallas as pl
from jax.experimental.pallas import tpu as pltpu
from jax.experimental.pallas import tpu_sc as plsc

NC, NS = 2, 16
NW = NC * NS
B = 16384
D = 64
BPW = B // NW
H1, H2 = 128, 64
CHUNK = 2048
IB = 128
NI = BPW // IB
LANES = 16


def _gather_body(uid_hbm, iid_hbm, utab_hbm, itab_hbm, ubias_hbm, ibias_hbm,
                 x_out, bsum_out,
                 uidx_v, iidx_v, urow_v, irow_v,
                 urows_v, irows_v, ubrows_v, ibrows_v, bsum_v, sem):
    wid = lax.axis_index("s") * NC + lax.axis_index("c")
    base = wid * BPW
    pltpu.sync_copy(uid_hbm.at[pl.ds(wid * NI, NI)], uidx_v)
    pltpu.sync_copy(iid_hbm.at[pl.ds(wid * NI, NI)], iidx_v)
    for j in range(NI):
        for k in range(IB // LANES):
            sl = pl.ds(k * LANES, LANES)
            urow_v[j, sl] = uidx_v[j, sl] >> 4
            irow_v[j, sl] = iidx_v[j, sl] >> 4
    cps = []
    for j in range(NI):
        sl = pl.ds(j * IB, IB)
        cps.append(pltpu.async_copy(utab_hbm.at[uidx_v.at[j]], urows_v.at[sl], sem))
        cps.append(pltpu.async_copy(itab_hbm.at[iidx_v.at[j]], irows_v.at[sl], sem))
        cps.append(pltpu.async_copy(ubias_hbm.at[urow_v.at[j]], ubrows_v.at[sl], sem))
        cps.append(pltpu.async_copy(ibias_hbm.at[irow_v.at[j]], ibrows_v.at[sl], sem))
    for cp in cps:
        cp.wait()
    for c in range(BPW // LANES):
        j, k = divmod(c, IB // LANES)
        sl = pl.ds(k * LANES, LANES)
        pos = lax.iota(jnp.int32, LANES) + c * LANES
        ubv16 = plsc.load_gather(ubrows_v, [pos, uidx_v[j, sl] & 15])
        ibv16 = plsc.load_gather(ibrows_v, [pos, iidx_v[j, sl] & 15])
        bsum_v[pl.ds(c * LANES, LANES)] = ubv16 + ibv16
    pltpu.sync_copy(urows_v, x_out.at[pl.ds(base, BPW), pl.ds(0, D)])
    pltpu.sync_copy(irows_v, x_out.at[pl.ds(base, BPW), pl.ds(D, D)])
    pltpu.sync_copy(bsum_v, bsum_out.at[pl.ds(base, BPW)])


@functools.cache
def _make_gather():
    return functools.partial(
        pl.kernel,
        out_type=[
            jax.ShapeDtypeStruct((B, 2 * D), jnp.float32),
            jax.ShapeDtypeStruct((B,), jnp.float32),
        ],
        mesh=plsc.VectorSubcoreMesh(core_axis_name="c", subcore_axis_name="s"),
        compiler_params=pltpu.CompilerParams(use_tc_tiling_on_sc=False,
                                             needs_layout_passes=False),
        scratch_types=[
            pltpu.VMEM((NI, IB), jnp.int32),
            pltpu.VMEM((NI, IB), jnp.int32),
            pltpu.VMEM((NI, IB), jnp.int32),
            pltpu.VMEM((NI, IB), jnp.int32),
            pltpu.VMEM((BPW, D), jnp.float32),
            pltpu.VMEM((BPW, D), jnp.float32),
            pltpu.VMEM((BPW, LANES), jnp.float32),
            pltpu.VMEM((BPW, LANES), jnp.float32),
            pltpu.VMEM((BPW,), jnp.float32),
            pltpu.SemaphoreType.DMA,
        ],
    )(_gather_body)


NB = NUM_ROWS = 1000001
PADW = 62501 * LANES
RCHUNK = 131072


def _repack_body(ub_ref, ib_ref, ubo_ref, ibo_ref):
    ubo_ref[...] = ub_ref[...]
    ibo_ref[...] = ib_ref[...]


def _repack(ubf, ibf):
    ngrid = (PADW + RCHUNK - 1) // RCHUNK
    return pl.pallas_call(
        _repack_body,
        grid=(ngrid,),
        in_specs=[pl.BlockSpec((RCHUNK,), lambda b: (b,)),
                  pl.BlockSpec((RCHUNK,), lambda b: (b,))],
        out_specs=[pl.BlockSpec((RCHUNK,), lambda b: (b,)),
                   pl.BlockSpec((RCHUNK,), lambda b: (b,))],
        out_shape=[jax.ShapeDtypeStruct((PADW,), jnp.float32),
                   jax.ShapeDtypeStruct((PADW,), jnp.float32)],
    )(ubf, ibf)


def _mlp_body(x_ref, b_ref, w1_ref, b1_ref,
              w2_ref, b2_ref, w3_ref, b3_ref, o_ref):
    h = jnp.dot(x_ref[...], w1_ref[...], preferred_element_type=jnp.float32)
    h = jnp.maximum(h + b1_ref[...], 0.0)
    h = jnp.maximum(
        jnp.dot(h, w2_ref[...], preferred_element_type=jnp.float32) + b2_ref[...],
        0.0)
    o = jnp.sum(h * w3_ref[...], axis=1, keepdims=True)
    o_ref[...] = o + b3_ref[...] + b_ref[...]


def _mlp(x, bsum, w1, b1, w2, b2, w3, b3):
    full = lambda b: (0, 0)
    return pl.pallas_call(
        _mlp_body,
        grid=(B // CHUNK,),
        in_specs=[
            pl.BlockSpec((CHUNK, 2 * D), lambda b: (b, 0)),
            pl.BlockSpec((CHUNK, 1), lambda b: (b, 0)),
            pl.BlockSpec((2 * D, H1), full),
            pl.BlockSpec((1, H1), full),
            pl.BlockSpec((H1, H2), full),
            pl.BlockSpec((1, H2), full),
            pl.BlockSpec((1, H2), full),
            pl.BlockSpec((1, 1), full),
        ],
        out_specs=pl.BlockSpec((CHUNK, 1), lambda b: (b, 0)),
        out_shape=jax.ShapeDtypeStruct((B, 1), jnp.float32),
    )(x, bsum, w1, b1, w2, b2, w3, b3)


def kernel(user_ids, item_ids, user_table, item_table, user_bias, item_bias,
           W1, b1, W2, b2, W3, b3):
    ubp, ibp = _repack(user_bias.reshape(-1), item_bias.reshape(-1))
    ubp = ubp.reshape(-1, LANES)
    ibp = ibp.reshape(-1, LANES)
    x, bsum = _make_gather()(
        user_ids.astype(jnp.int32).reshape(B // IB, IB),
        item_ids.astype(jnp.int32).reshape(B // IB, IB),
        user_table, item_table, ubp, ibp)
    out = _mlp(x, bsum.reshape(B, 1), W1.T, b1.reshape(1, H1),
               W2.T, b2.reshape(1, H2), W3, b3.reshape(1, 1))
    return out[:, 0]
